# baseline (speedup 1.0000x reference)
import functools
import math

import jax
import jax.numpy as jnp
from jax import lax
from jax.experimental import pallas as pl
from jax.experimental.pallas import tpu as pltpu
from jax.experimental.pallas import tpu_sc as plsc

_N_POS = 1000000
_D = 64
_NB = math.ceil(math.log2(_N_POS))
_LO = 10
_HI = _NB - _LO
_T2 = (1 << _LO) + (1 << _HI)

_NC = 2
_NS = 16
_NW = _NC * _NS
_L = 16

_CHUNK = 64


def _t2_body(tlo_ref, thi_ref, out_ref):
    n = 1 << _LO
    v = lax.broadcasted_iota(jnp.int32, (n, 32), 0)
    b = lax.broadcasted_iota(jnp.int32, (n, 32), 1)
    bits = ((v >> b) & 1).astype(jnp.float32)
    out_ref[:, :_D] = lax.dot(bits, tlo_ref[...],
                              precision=lax.Precision.HIGHEST)
    out_ref[:, _D:] = lax.dot(bits, thi_ref[...],
                              precision=lax.Precision.HIGHEST)


def _build_table2(table, interpret=False):
    tlo = jnp.zeros((32, _D), jnp.float32).at[:_LO].set(table[:_LO])
    thi = jnp.zeros((32, _D), jnp.float32).at[:_HI].set(table[_LO:_NB])
    return pl.pallas_call(
        _t2_body,
        out_shape=jax.ShapeDtypeStruct((1 << _LO, 2 * _D), jnp.float32),
        interpret=interpret,
    )(tlo, thi)


def _sc_embed(x_flat, t2):
    t = x_flat.shape[0]
    per_w = t // _NW
    n_chunks = per_w // _CHUNK
    mesh = plsc.VectorSubcoreMesh(core_axis_name="c", subcore_axis_name="s")

    @functools.partial(
        pl.kernel, mesh=mesh,
        out_type=jax.ShapeDtypeStruct((t * _D,), jnp.float32),
        scratch_types=[
            pltpu.VMEM((_CHUNK,), jnp.int32),
            pltpu.VMEM((_CHUNK,), jnp.int32),
            pltpu.VMEM((_CHUNK,), jnp.int32),
            pltpu.VMEM((_CHUNK, 2 * _D), jnp.float32),
            pltpu.VMEM((_CHUNK, 2 * _D), jnp.float32),
            pltpu.VMEM((_CHUNK * _D,), jnp.float32),
            pltpu.SemaphoreType.DMA,
        ],
    )
    def k(x_hbm, t2_hbm, out_hbm, x_c, ilo, ihi, ba, bb, bs, sem):
        wid = lax.axis_index("s") * _NC + lax.axis_index("c")
        base = wid * per_w

        def step(c, _):
            off = c * _CHUNK
            pltpu.sync_copy(x_hbm.at[pl.ds(base + off, _CHUNK)], x_c)
            for i in range(_CHUNK // _L):
                s = pl.ds(i * _L, _L)
                v = x_c[s]
                ilo[s] = v & ((1 << _LO) - 1)
                ihi[s] = v >> _LO
            ga = pltpu.async_copy(t2_hbm.at[ilo], ba, sem)
            gb = pltpu.async_copy(t2_hbm.at[ihi], bb, sem)
            ga.wait()
            gb.wait()

            for r in range(_CHUNK):
                for j in range(_D // _L):
                    bs[pl.ds(r * _D + j * _L, _L)] = (
                        ba[r, pl.ds(j * _L, _L)]
                        + bb[r, pl.ds(_D + j * _L, _L)])
            pltpu.sync_copy(
                bs, out_hbm.at[pl.ds((base + off) * _D, _CHUNK * _D)])
            return 0

        lax.fori_loop(0, n_chunks, step, 0)

    return k(x_flat, t2).reshape(t, _D)


def kernel(x, table):
    x_flat = x.reshape(-1)
    t2 = _build_table2(table)
    return _sc_embed(x_flat, t2)


# speedup vs baseline: 1.0445x; 1.0445x over previous
import functools
import math

import jax
import jax.numpy as jnp
from jax import lax
from jax.experimental import pallas as pl
from jax.experimental.pallas import tpu as pltpu
from jax.experimental.pallas import tpu_sc as plsc

_N_POS = 1000000
_D = 64
_NB = math.ceil(math.log2(_N_POS))
_LO = 10
_HI = _NB - _LO
_T2 = (1 << _LO) + (1 << _HI)

_NC = 2
_NS = 16
_NW = _NC * _NS
_L = 16

_CHUNK = 128


def _t2_body(tlo_ref, thi_ref, out_ref):
    n = 1 << _LO
    v = lax.broadcasted_iota(jnp.int32, (n, 32), 0)
    b = lax.broadcasted_iota(jnp.int32, (n, 32), 1)
    bits = ((v >> b) & 1).astype(jnp.float32)
    out_ref[:, :_D] = lax.dot(bits, tlo_ref[...],
                              precision=lax.Precision.HIGHEST)
    out_ref[:, _D:] = lax.dot(bits, thi_ref[...],
                              precision=lax.Precision.HIGHEST)


def _build_table2(table, interpret=False):
    tlo = jnp.zeros((32, _D), jnp.float32).at[:_LO].set(table[:_LO])
    thi = jnp.zeros((32, _D), jnp.float32).at[:_HI].set(table[_LO:_NB])
    return pl.pallas_call(
        _t2_body,
        out_shape=jax.ShapeDtypeStruct((1 << _LO, 2 * _D), jnp.float32),
        interpret=interpret,
    )(tlo, thi)


def _sc_embed(x_flat, t2):
    t = x_flat.shape[0]
    per_w = t // _NW
    n_chunks = per_w // _CHUNK
    mesh = plsc.VectorSubcoreMesh(core_axis_name="c", subcore_axis_name="s")

    @functools.partial(
        pl.kernel, mesh=mesh,
        out_type=jax.ShapeDtypeStruct((t * _D,), jnp.float32),
        scratch_types=[
            pltpu.VMEM((_CHUNK,), jnp.int32),
            pltpu.VMEM((_CHUNK,), jnp.int32),
            pltpu.VMEM((_CHUNK,), jnp.int32),
            pltpu.VMEM((_CHUNK, 2 * _D), jnp.float32),
            pltpu.VMEM((_CHUNK, 2 * _D), jnp.float32),
            pltpu.VMEM((_CHUNK * _D,), jnp.float32),
            pltpu.SemaphoreType.DMA,
        ],
    )
    def k(x_hbm, t2_hbm, out_hbm, x_c, ilo, ihi, ba, bb, bs, sem):
        wid = lax.axis_index("s") * _NC + lax.axis_index("c")
        base = wid * per_w

        def step(c, _):
            off = c * _CHUNK
            pltpu.sync_copy(x_hbm.at[pl.ds(base + off, _CHUNK)], x_c)
            for i in range(_CHUNK // _L):
                s = pl.ds(i * _L, _L)
                v = x_c[s]
                ilo[s] = v & ((1 << _LO) - 1)
                ihi[s] = v >> _LO
            ga = pltpu.async_copy(t2_hbm.at[ilo], ba, sem)
            gb = pltpu.async_copy(t2_hbm.at[ihi], bb, sem)
            ga.wait()
            gb.wait()

            for r in range(_CHUNK):
                for j in range(_D // _L):
                    bs[pl.ds(r * _D + j * _L, _L)] = (
                        ba[r, pl.ds(j * _L, _L)]
                        + bb[r, pl.ds(_D + j * _L, _L)])
            pltpu.sync_copy(
                bs, out_hbm.at[pl.ds((base + off) * _D, _CHUNK * _D)])
            return 0

        lax.fori_loop(0, n_chunks, step, 0)

    return k(x_flat, t2).reshape(t, _D)


def kernel(x, table):
    x_flat = x.reshape(-1)
    t2 = _build_table2(table)
    return _sc_embed(x_flat, t2)
